# baseline (speedup 1.0000x reference)
import functools

import jax
import jax.numpy as jnp
from jax import lax
from jax.experimental import pallas as pl
from jax.experimental.pallas import tpu as pltpu
from jax.experimental.pallas import tpu_sc as plsc

N = 100000
D = 32
HD = D // 2
NS = 16
CH = 704
NCH = 144
TPS = CH * NCH
E_PAD = TPS * NS


def _layer_body(scale, x_hbm, s_hbm, sd_hbm, w_hbm, xo_hbm, so_hbm,
                acc, sdvA, sdvB, dsA, dsB, rowsA, rowsB, gsem, isem, ssem):
    c = lax.axis_index("c")
    sid = lax.axis_index("s")
    base = c * N
    z16 = jnp.zeros((16,), jnp.float32)
    sdbufs = (sdvA, sdvB)
    dbufs = (dsA, dsB)
    rbufs = (rowsA, rowsB)

    def zbody(e, carry):
        rowsA[e, pl.ds(0, 16)] = z16
        return carry
    lax.fori_loop(0, CH, zbody, 0)
    for t in range(9):
        b = sid + 16 * t
        @pl.when(b <= 141)
        def _():
            pltpu.sync_copy(rowsA.at[pl.ds(0, CH)], acc.at[pl.ds(b * CH, CH)])
    @pl.when(sid == 1)
    def _():
        pltpu.sync_copy(rowsA.at[pl.ds(0, 32)], acc.at[pl.ds(142 * CH, 32)])
    plsc.subcore_barrier()

    def fetch(k, bi):
        pltpu.make_async_copy(sd_hbm.at[c, sid * NCH + k], sdbufs[bi], isem).start()

    def wait_fetch(bi):
        pltpu.make_async_copy(sd_hbm.at[0, 0], sdbufs[bi], isem).wait()

    def fire_gather(bi):
        pltpu.make_async_copy(x_hbm.at[sdbufs[bi].at[0]], rbufs[bi], gsem).start()

    def wait_gather(bi):
        pltpu.make_async_copy(x_hbm.at[sdbufs[bi].at[0]], rbufs[bi], gsem).wait()

    def fire_scatter(bi):
        pltpu.make_async_copy(rbufs[bi], acc.at[dbufs[bi]], ssem).start(add=True)

    def wait_scatter(bi):
        pltpu.make_async_copy(rbufs[bi], acc.at[dbufs[bi]], ssem).wait()

    def do_chunk(k, p):
        sdp = sdbufs[p]
        dsp = dbufs[p]
        rwp = rbufs[p]
        @pl.when(k >= 1)
        def _():
            wait_scatter(1 - p)
        @pl.when(k + 1 < NCH)
        def _():
            wait_fetch(1 - p)
            fire_gather(1 - p)

        wait_gather(p)

        def wmul(j, carry2):
            js = pl.ds(j * 16, 16)
            wgrp = plsc.bitcast(sdp[2, js], jnp.float32)
            dsp[js] = sdp[1, js]
            e0 = j * 16
            for i in range(16):
                w = wgrp[i]
                rwp[e0 + i, pl.ds(0, 16)] = rwp[e0 + i, pl.ds(0, 16)] * w
            return carry2
        lax.fori_loop(0, CH // 16, wmul, 0)

        @pl.when(k + 2 < NCH)
        def _():
            fetch(k + 2, p)
        fire_scatter(p)

    fetch(0, 0)
    wait_fetch(0)
    fire_gather(0)
    fetch(1, 1)

    def dbl(kk, carry):
        do_chunk(2 * kk, 0)
        do_chunk(2 * kk + 1, 1)
        return carry
    lax.fori_loop(0, NCH // 2, dbl, 0)
    wait_scatter(1)
    plsc.subcore_barrier()

    WB = CH // 2

    def wout(o, n):
        pltpu.sync_copy(acc.at[pl.ds(o, n)], rowsA.at[pl.ds(0, n)])
        pltpu.sync_copy(s_hbm.at[pl.ds(base + o, n)], rowsA.at[pl.ds(WB, n)])

        def sadd(e, carry):
            a0 = rowsA[e, pl.ds(0, 16)] + rowsA[WB + e, pl.ds(0, 16)]
            if scale != 1.0:
                a0 = a0 * scale
            rowsA[WB + e, pl.ds(0, 16)] = a0
            return carry
        lax.fori_loop(0, n, sadd, 0)
        pltpu.sync_copy(rowsA.at[pl.ds(0, n)], xo_hbm.at[pl.ds(base + o, n)])
        pltpu.sync_copy(rowsA.at[pl.ds(WB, n)], so_hbm.at[pl.ds(base + o, n)])

    for t in range(18):
        b = sid + 16 * t
        @pl.when(b <= 283)
        def _():
            wout(b * WB, WB)
    @pl.when(sid == 5)
    def _():
        wout(284 * WB, 32)


def _make_layer(scale):
    return pl.kernel(
        functools.partial(_layer_body, scale),
        out_type=(jax.ShapeDtypeStruct((2 * N, HD), jnp.float32),
                  jax.ShapeDtypeStruct((2 * N, HD), jnp.float32)),
        mesh=plsc.VectorSubcoreMesh(core_axis_name="c", subcore_axis_name="s"),
        compiler_params=pltpu.CompilerParams(use_tc_tiling_on_sc=False, needs_layout_passes=False),
        scratch_types=[
            pltpu.VMEM_SHARED((N, HD), jnp.float32),
            pltpu.VMEM((3, CH), jnp.int32),
            pltpu.VMEM((3, CH), jnp.int32),
            pltpu.VMEM((CH,), jnp.int32),
            pltpu.VMEM((CH,), jnp.int32),
            pltpu.VMEM((CH, HD), jnp.float32),
            pltpu.VMEM((CH, HD), jnp.float32),
            pltpu.SemaphoreType.DMA,
            pltpu.SemaphoreType.DMA,
            pltpu.SemaphoreType.DMA,
        ],
    )


_layer_mid = _make_layer(1.0)
_layer_last = _make_layer(0.25)


def kernel(emb, edge_index, edge_weight):
    e = edge_index.shape[1]
    pad = E_PAD - e
    src = jnp.concatenate([edge_index[0], jnp.zeros((pad,), jnp.int32)])
    dst = jnp.concatenate([edge_index[1], jnp.zeros((pad,), jnp.int32)])
    w = jnp.concatenate([edge_weight, jnp.zeros((pad,), jnp.float32)])
    srcs = src.reshape(-1, CH)
    dsts = dst.reshape(-1, CH)
    wbits = jax.lax.bitcast_convert_type(w, jnp.int32).reshape(-1, CH)
    sd = jnp.stack([jnp.stack([srcs, dsts, wbits], axis=1),
                    jnp.stack([srcs + N, dsts, wbits], axis=1)])
    x = jnp.concatenate([emb[:, :HD], emb[:, HD:]], axis=0)
    s = x
    x, s = _layer_mid(x, s, sd, w)
    x, s = _layer_mid(x, s, sd, w)
    x, s = _layer_last(x, s, sd, w)
    return jnp.concatenate([s[:N], s[N:]], axis=1)
